# baseline (speedup 1.0000x reference)
import functools

import jax
import jax.numpy as jnp
from jax import lax
from jax.experimental import pallas as pl
from jax.experimental.pallas import tpu as pltpu
from jax.experimental.pallas import tpu_sc as plsc

_INFO = plsc.get_sparse_core_info()
_NC = _INFO.num_cores
_NS = _INFO.num_subcores
_NW = _NC * _NS
_L = 16
_PR = 7680
_B = 16


def _body(n_pop, n_per_w, d, table_hbm, idx_hbm, out_hbm,
          idx_v, cidx_v, cpos_v, cidx16_v, tbuf_v, rows_v, piece_sh, sem):
    cid = lax.axis_index("c")
    sid = lax.axis_index("s")
    wid = sid * _NC + cid
    base = wid * n_per_w
    npiece = -(-n_pop // _PR)
    stage_rows = _PR // _NS

    pltpu.sync_copy(idx_hbm.at[pl.ds(base, n_per_w)], idx_v)

    iota = lax.iota(jnp.int32, _L)
    nvec = n_per_w // _L

    def piece_body(p, carry):
        parity = 0
        lo = p * _PR
        pbase = jnp.minimum(lo, n_pop - _PR)
        src0 = pbase + sid * stage_rows
        pltpu.sync_copy(
            table_hbm.at[pl.ds(src0, stage_rows)],
            piece_sh.at[parity, pl.ds(sid * stage_rows, stage_rows)])
        plsc.subcore_barrier()

        def scan_vec(k, cnt):
            iv = idx_v[pl.ds(k * _L, _L)]
            m = (iv >= lo) & (iv < lo + _PR)
            rel = iv - pbase
            pos = iota + k * _L
            plsc.store_compressed(cidx_v.at[pl.ds(cnt, _L)], rel, mask=m)
            plsc.store_compressed(cpos_v.at[pl.ds(cnt, _L)], pos, mask=m)
            return cnt + jnp.sum(m.astype(jnp.int32))

        cnt = lax.fori_loop(0, nvec, scan_vec, jnp.int32(0))
        cidx_v[pl.ds(cnt, _L)] = jnp.zeros((_L,), jnp.int32)

        def batch_body(b, carry2):
            cidx16_v[...] = cidx_v[pl.ds(b * _B, _B)]
            pltpu.async_copy(
                piece_sh.at[parity].at[cidx16_v], tbuf_v, sem).wait()
            cpos16 = cpos_v[pl.ds(b * _B, _B)]
            valid = (b * _B + iota) < cnt

            def col_body(c, carry3):
                cvec = jnp.zeros((_L,), jnp.int32) + c
                x = plsc.load_gather(tbuf_v, [iota, cvec])
                plsc.store_scatter(rows_v, [cpos16, cvec], x, mask=valid)
                return carry3

            lax.fori_loop(0, d, col_body, jnp.int32(0))
            return carry2

        nb = (cnt + (_B - 1)) // _B
        lax.fori_loop(0, nb, batch_body, jnp.int32(0))

        plsc.subcore_barrier()
        return carry

    lax.fori_loop(0, npiece, piece_body, jnp.int32(0))
    pltpu.sync_copy(rows_v, out_hbm.at[wid])


def kernel(full_x, indices):
    n = indices.shape[0]
    n_pop, d = full_x.shape
    n_per_w = n // _NW
    idx = indices.astype(jnp.int32)

    body = functools.partial(_body, n_pop, n_per_w, d)
    out = pl.kernel(
        body,
        out_type=jax.ShapeDtypeStruct((_NW, n_per_w, d), jnp.float32),
        mesh=plsc.VectorSubcoreMesh(core_axis_name="c", subcore_axis_name="s"),
        scratch_types=[
            pltpu.VMEM((n_per_w,), jnp.int32),
            pltpu.VMEM((n_per_w + _L,), jnp.int32),
            pltpu.VMEM((n_per_w + _L,), jnp.int32),
            pltpu.VMEM((_B,), jnp.int32),
            pltpu.VMEM((_B, d), jnp.float32),
            pltpu.VMEM((n_per_w, d), jnp.float32),
            pltpu.VMEM_SHARED((1, _PR, d), jnp.float32),
            pltpu.SemaphoreType.DMA,
        ],
        compiler_params=pltpu.CompilerParams(
            use_tc_tiling_on_sc=True, needs_layout_passes=False),
    )(full_x, idx)
    return out.reshape(n, d)


# speedup vs baseline: 1.3655x; 1.3655x over previous
import functools

import jax
import jax.numpy as jnp
from jax import lax
from jax.experimental import pallas as pl
from jax.experimental.pallas import tpu as pltpu
from jax.experimental.pallas import tpu_sc as plsc

_INFO = plsc.get_sparse_core_info()
_NC = _INFO.num_cores
_NS = _INFO.num_subcores
_NW = _NC * _NS
_L = 16
_CR = 160
_GC = 128


def _repack_body(half, d, nchunk, table_hbm, dense_hbm,
                 bufa, bufb, big, rsem, wsem):
    wid = lax.axis_index("s") * _NC + lax.axis_index("c")
    nloop = -(-nchunk // _NW)

    def reads(j, par):
        r0 = j * _CR
        pltpu.async_copy(
            table_hbm.at[pl.ds(r0, _CR)], bufa.at[par], rsem)
        pltpu.async_copy(
            table_hbm.at[pl.ds(half + r0, _CR)], bufb.at[par], rsem)

    def wait_read(par):
        for ref in (bufa, bufb):
            pltpu.make_async_copy(
                table_hbm.at[pl.ds(0, _CR)], ref.at[par], rsem).wait()

    reads(wid, 0)

    def chunk_loop(i, carry):
        j = wid + i * _NW

        @pl.when(j < nchunk)
        def _():
            par = i & 1
            wait_read(par)

            @pl.when(j + _NW < nchunk)
            def _():
                reads(j + _NW, 1 - par)

            @pl.when(i >= 2)
            def _():
                pltpu.make_async_copy(
                    big.at[par], dense_hbm.at[pl.ds(0, _CR)], wsem).wait()

            def move_row(r, carry2):
                for cg in range(d // _L):
                    big[par, r, pl.ds(cg * _L, _L)] = (
                        bufa[par, r, pl.ds(cg * _L, _L)])
                    big[par, r, pl.ds(d + cg * _L, _L)] = (
                        bufb[par, r, pl.ds(cg * _L, _L)])
                return carry2

            lax.fori_loop(0, _CR, move_row, jnp.int32(0))
            pltpu.async_copy(
                big.at[par], dense_hbm.at[pl.ds(j * _CR, _CR)], wsem)

        return carry

    lax.fori_loop(0, nloop, chunk_loop, jnp.int32(0))
    for _ in range(2):
        pltpu.make_async_copy(
            big.at[0], dense_hbm.at[pl.ds(0, _CR)], wsem).wait()


def _gather_body(half, n_per_w, d, dense_hbm, idx_hbm, out_hbm,
                 idx_v, blk_v, recs_v, rows_v, sem):
    wid = lax.axis_index("s") * _NC + lax.axis_index("c")
    base = wid * n_per_w
    nchunk = n_per_w // _GC
    pltpu.sync_copy(idx_hbm.at[pl.ds(base, n_per_w)], idx_v)

    for k in range(n_per_w // _L):
        iv = idx_v[pl.ds(k * _L, _L)]
        v = jnp.where(iv >= half, iv - half, iv)
        blk_v[k * _L // _GC, pl.ds(k * _L % _GC, _L)] = v

    iota = lax.iota(jnp.int32, _L)
    copies = [pltpu.async_copy(dense_hbm.at[blk_v.at[0]], recs_v.at[0], sem)]
    for j in range(nchunk):
        par = j & 1
        copies.pop(0).wait()
        if j + 1 < nchunk:
            copies.append(pltpu.async_copy(
                dense_hbm.at[blk_v.at[j + 1]], recs_v.at[1 - par], sem))
        pvec = jnp.zeros((_L,), jnp.int32) + par

        def grp_body(g, carry, j=j, par=par, pvec=pvec):
            iv = idx_v[pl.ds(j * _GC + g * _L, _L)]
            hvec = jnp.where(iv >= half, d, 0)
            rvec = iota + g * _L

            def col_body(c, carry2):
                cvec = jnp.zeros((_L,), jnp.int32) + c
                x = plsc.load_gather(recs_v, [pvec, rvec, hvec + cvec])
                plsc.store_scatter(rows_v, [rvec + j * _GC, cvec], x)
                return carry2

            lax.fori_loop(0, d, col_body, jnp.int32(0))
            return carry

        lax.fori_loop(0, _GC // _L, grp_body, jnp.int32(0))
    pltpu.sync_copy(rows_v, out_hbm.at[wid])


def kernel(full_x, indices):
    n = indices.shape[0]
    n_pop, d = full_x.shape
    half = n_pop // 2
    n_per_w = n // _NW
    nchunk = half // _CR
    idx = indices.astype(jnp.int32)
    mesh = plsc.VectorSubcoreMesh(core_axis_name="c", subcore_axis_name="s")
    params = pltpu.CompilerParams(
        use_tc_tiling_on_sc=True, needs_layout_passes=False)

    repack = pl.kernel(
        functools.partial(_repack_body, half, d, nchunk),
        out_type=jax.ShapeDtypeStruct((half, 2 * d), jnp.float32),
        mesh=mesh,
        scratch_types=[
            pltpu.VMEM((2, _CR, d), jnp.float32),
            pltpu.VMEM((2, _CR, d), jnp.float32),
            pltpu.VMEM((2, _CR, 2 * d), jnp.float32),
            pltpu.SemaphoreType.DMA,
            pltpu.SemaphoreType.DMA,
        ],
        compiler_params=params,
    )
    dense = repack(full_x)

    gather = pl.kernel(
        functools.partial(_gather_body, half, n_per_w, d),
        out_type=jax.ShapeDtypeStruct((_NW, n_per_w, d), jnp.float32),
        mesh=mesh,
        scratch_types=[
            pltpu.VMEM((n_per_w,), jnp.int32),
            pltpu.VMEM((n_per_w // _GC, _GC), jnp.int32),
            pltpu.VMEM((2, _GC, 2 * d), jnp.float32),
            pltpu.VMEM((n_per_w, d), jnp.float32),
            pltpu.SemaphoreType.DMA,
        ],
        compiler_params=params,
    )
    out = gather(dense, idx)
    return out.reshape(n, d)


# speedup vs baseline: 1.8623x; 1.3638x over previous
import functools

import jax
import jax.numpy as jnp
from jax import lax
from jax.experimental import pallas as pl
from jax.experimental.pallas import tpu as pltpu
from jax.experimental.pallas import tpu_sc as plsc

_INFO = plsc.get_sparse_core_info()
_NC = _INFO.num_cores
_NS = _INFO.num_subcores
_NW = _NC * _NS
_L = 16
_CR = 160
_GC = 128


def _repack_body(half, d, nchunk, table_hbm, dense_hbm,
                 bufa, bufb, big, rsem, wsem):
    wid = lax.axis_index("s") * _NC + lax.axis_index("c")
    nloop = -(-nchunk // _NW)

    def reads(j, par):
        r0 = j * _CR
        pltpu.async_copy(
            table_hbm.at[pl.ds(r0, _CR)], bufa.at[par], rsem)
        pltpu.async_copy(
            table_hbm.at[pl.ds(half + r0, _CR)], bufb.at[par], rsem)

    def wait_read(par):
        for ref in (bufa, bufb):
            pltpu.make_async_copy(
                table_hbm.at[pl.ds(0, _CR)], ref.at[par], rsem).wait()

    reads(wid, 0)

    def chunk_loop(i, carry):
        j = wid + i * _NW

        @pl.when(j < nchunk)
        def _():
            par = i & 1
            wait_read(par)

            @pl.when(j + _NW < nchunk)
            def _():
                reads(j + _NW, 1 - par)

            @pl.when(i >= 2)
            def _():
                pltpu.make_async_copy(
                    big.at[par], dense_hbm.at[pl.ds(0, _CR)], wsem).wait()

            def move_row(r, carry2):
                for cg in range(d // _L):
                    big[par, r, pl.ds(cg * _L, _L)] = (
                        bufa[par, r, pl.ds(cg * _L, _L)])
                    big[par, r, pl.ds(d + cg * _L, _L)] = (
                        bufb[par, r, pl.ds(cg * _L, _L)])
                return carry2

            lax.fori_loop(0, _CR, move_row, jnp.int32(0))
            pltpu.async_copy(
                big.at[par], dense_hbm.at[pl.ds(j * _CR, _CR)], wsem)

        return carry

    lax.fori_loop(0, nloop, chunk_loop, jnp.int32(0))
    for _ in range(2):
        pltpu.make_async_copy(
            big.at[0], dense_hbm.at[pl.ds(0, _CR)], wsem).wait()


def _gather_body(half, n_per_w, d, dense_hbm, idx_hbm, out_hbm,
                 idx_v, blk_v, recs_v, rows_v, sem):
    wid = lax.axis_index("s") * _NC + lax.axis_index("c")
    base = wid * n_per_w
    nchunk = n_per_w // _GC
    pltpu.sync_copy(idx_hbm.at[pl.ds(base, n_per_w)], idx_v)

    for k in range(n_per_w // _L):
        iv = idx_v[pl.ds(k * _L, _L)]
        blk_v[k * _L // _GC, pl.ds(k * _L % _GC, _L)] = iv >> 1

    iota = lax.iota(jnp.int32, _L)
    copies = [pltpu.async_copy(dense_hbm.at[blk_v.at[0]], recs_v.at[0], sem)]
    for j in range(nchunk):
        par = j & 1
        copies.pop(0).wait()
        if j + 1 < nchunk:
            copies.append(pltpu.async_copy(
                dense_hbm.at[blk_v.at[j + 1]], recs_v.at[1 - par], sem))
        pvec = jnp.zeros((_L,), jnp.int32) + par

        def grp_body(g, carry, j=j, par=par, pvec=pvec):
            iv = idx_v[pl.ds(j * _GC + g * _L, _L)]
            hvec = (iv & 1) * d
            rvec = iota + g * _L

            def col_body(c, carry2):
                cvec = jnp.zeros((_L,), jnp.int32) + c
                x = plsc.load_gather(recs_v, [pvec, rvec, hvec + cvec])
                plsc.store_scatter(rows_v, [rvec + j * _GC, cvec], x)
                return carry2

            lax.fori_loop(0, d, col_body, jnp.int32(0))
            return carry

        lax.fori_loop(0, _GC // _L, grp_body, jnp.int32(0))
    pltpu.sync_copy(rows_v, out_hbm.at[wid])


def kernel(full_x, indices):
    n = indices.shape[0]
    n_pop, d = full_x.shape
    half = n_pop // 2
    n_per_w = n // _NW
    nchunk = half // _CR
    idx = indices.astype(jnp.int32)
    mesh = plsc.VectorSubcoreMesh(core_axis_name="c", subcore_axis_name="s")
    params = pltpu.CompilerParams(
        use_tc_tiling_on_sc=True, needs_layout_passes=False)

    dense = full_x.reshape(half, 2 * d)

    gather = pl.kernel(
        functools.partial(_gather_body, half, n_per_w, d),
        out_type=jax.ShapeDtypeStruct((_NW, n_per_w, d), jnp.float32),
        mesh=mesh,
        scratch_types=[
            pltpu.VMEM((n_per_w,), jnp.int32),
            pltpu.VMEM((n_per_w // _GC, _GC), jnp.int32),
            pltpu.VMEM((2, _GC, 2 * d), jnp.float32),
            pltpu.VMEM((n_per_w, d), jnp.float32),
            pltpu.SemaphoreType.DMA,
        ],
        compiler_params=params,
    )
    out = gather(dense, idx)
    return out.reshape(n, d)


# speedup vs baseline: 2.5235x; 1.3551x over previous
import functools

import jax
import jax.numpy as jnp
from jax import lax
from jax.experimental import pallas as pl
from jax.experimental.pallas import tpu as pltpu
from jax.experimental.pallas import tpu_sc as plsc

_INFO = plsc.get_sparse_core_info()
_NC = _INFO.num_cores
_NS = _INFO.num_subcores
_NW = _NC * _NS
_L = 16


def _body(n_per_w, d, table_hbm, idx_hbm, out_hbm, idx_v, rows_v, sem, sem2):
    wid = lax.axis_index("s") * _NC + lax.axis_index("c")
    base = wid * n_per_w
    nh = n_per_w // 2
    pltpu.sync_copy(idx_hbm.at[pl.ds(base, n_per_w)], idx_v)

    def grp_body(g, carry):
        ivec = idx_v[pl.ds(g * _L, _L)]
        for lane in range(_L):
            i = ivec[lane]
            r = g * _L + lane
            pltpu.async_copy(
                table_hbm.at[pl.ds(i, 1)], rows_v.at[pl.ds(r, 1)], sem)
        return carry

    def grp_body2(g, carry):
        ivec = idx_v[pl.ds(nh + g * _L, _L)]
        for lane in range(_L):
            i = ivec[lane]
            r = nh + g * _L + lane
            pltpu.async_copy(
                table_hbm.at[pl.ds(i, 1)],
                out_hbm.at[wid, pl.ds(r, 1)], sem2)
        return carry

    lax.fori_loop(0, nh // _L, grp_body2, 0)
    lax.fori_loop(0, nh // _L, grp_body, 0)

    pltpu.make_async_copy(table_hbm.at[pl.ds(0, nh)], rows_v,
                          sem).wait()
    pltpu.make_async_copy(table_hbm.at[pl.ds(0, nh)],
                          out_hbm.at[wid, pl.ds(nh, nh)], sem2).wait()
    pltpu.sync_copy(rows_v, out_hbm.at[wid, pl.ds(0, nh)])


def kernel(full_x, indices):
    n = indices.shape[0]
    d = full_x.shape[1]
    n_per_w = n // _NW
    idx = indices.astype(jnp.int32)

    body = functools.partial(_body, n_per_w, d)
    out = pl.kernel(
        body,
        out_type=jax.ShapeDtypeStruct((_NW, n_per_w, d), jnp.float32),
        mesh=plsc.VectorSubcoreMesh(core_axis_name="c", subcore_axis_name="s"),
        scratch_types=[
            pltpu.VMEM((n_per_w,), jnp.int32),
            pltpu.VMEM((n_per_w // 2, d), jnp.float32),
            pltpu.SemaphoreType.DMA,
            pltpu.SemaphoreType.DMA,
        ],
        compiler_params=pltpu.CompilerParams(
            use_tc_tiling_on_sc=True, needs_layout_passes=False),
    )(full_x, idx)
    return out.reshape(n, d)


# speedup vs baseline: 3.3824x; 1.3403x over previous
import functools

import jax
import jax.numpy as jnp
from jax import lax
from jax.experimental import pallas as pl
from jax.experimental.pallas import tpu as pltpu
from jax.experimental.pallas import tpu_sc as plsc

_INFO = plsc.get_sparse_core_info()
_NC = _INFO.num_cores
_NS = _INFO.num_subcores
_NW = _NC * _NS
_L = 16


def _body(n_per_w, d, table_hbm, idx_hbm, out_hbm, idx_v, rows_v, sem):
    wid = lax.axis_index("s") * _NC + lax.axis_index("c")
    base = wid * n_per_w
    pltpu.sync_copy(idx_hbm.at[pl.ds(base, n_per_w)], idx_v)

    def grp_body(g, carry):
        ivec = idx_v[pl.ds(g * _L, _L)]
        for lane in range(_L):
            i = ivec[lane]
            r = g * _L + lane
            pltpu.async_copy(
                table_hbm.at[pl.ds(i, 1)], rows_v.at[pl.ds(r, 1)], sem)
        return carry

    lax.fori_loop(0, n_per_w // _L, grp_body, 0)

    pltpu.make_async_copy(table_hbm.at[pl.ds(0, n_per_w)], rows_v, sem).wait()
    pltpu.sync_copy(rows_v, out_hbm.at[wid])


def kernel(full_x, indices):
    n = indices.shape[0]
    d = full_x.shape[1]
    n_per_w = n // _NW
    idx = indices.astype(jnp.int32)

    body = functools.partial(_body, n_per_w, d)
    out = pl.kernel(
        body,
        out_type=jax.ShapeDtypeStruct((_NW, n_per_w, d), jnp.float32),
        mesh=plsc.VectorSubcoreMesh(core_axis_name="c", subcore_axis_name="s"),
        scratch_types=[
            pltpu.VMEM((n_per_w,), jnp.int32),
            pltpu.VMEM((n_per_w, d), jnp.float32),
            pltpu.SemaphoreType.DMA,
        ],
        compiler_params=pltpu.CompilerParams(
            use_tc_tiling_on_sc=True, needs_layout_passes=False),
    )(full_x, idx)
    return out.reshape(n, d)
